# Initial kernel scaffold; baseline (speedup 1.0000x reference)
import jax
import jax.numpy as jnp
from jax.experimental import pallas as pl


def kernel(x, W):
    raise NotImplementedError("write your pallas kernel here")


# speedup vs baseline: 3.2459x; 3.2459x over previous
import jax
import jax.numpy as jnp
from jax.experimental import pallas as pl
from jax.experimental.pallas import tpu as pltpu

_NUM_EXPERTS = 64
_TOP_K = 2
_N_TOKENS = 16384
_N_EMBD = 2048
_BLOCK = 512
_GRID = _N_TOKENS // _BLOCK


def _router_kernel(x_ref, wt_ref, scores_ref, idx_ref, aux_ref, acc_ref):
    step = pl.program_id(0)

    @pl.when(step == 0)
    def _init():
        acc_ref[...] = jnp.zeros_like(acc_ref)

    logits = jnp.dot(x_ref[...], wt_ref[...], preferred_element_type=jnp.float32)
    m = jnp.max(logits, axis=1, keepdims=True)
    e = jnp.exp(logits - m)
    denom = jnp.sum(e, axis=1, keepdims=True)
    s = e / denom

    iota = jax.lax.broadcasted_iota(jnp.int32, s.shape, 1)
    m1 = jnp.max(s, axis=1, keepdims=True)
    i1 = jnp.min(jnp.where(s == m1, iota, _NUM_EXPERTS), axis=1, keepdims=True)
    masked = jnp.where(iota == i1, -jnp.inf, s)
    m2 = jnp.max(masked, axis=1, keepdims=True)
    i2 = jnp.min(jnp.where(masked == m2, iota, _NUM_EXPERTS), axis=1, keepdims=True)

    scores_ref[:, 0:1] = m1
    scores_ref[:, 1:2] = m2
    idx_ref[:, 0:1] = i1
    idx_ref[:, 1:2] = i2

    colsum = jnp.sum(s, axis=0)[None, :]
    counts = jnp.sum(
        (iota == i1).astype(jnp.float32) + (iota == i2).astype(jnp.float32),
        axis=0,
    )[None, :]
    acc_ref[0:1, :] += colsum
    acc_ref[1:2, :] += counts

    @pl.when(step == _GRID - 1)
    def _finish():
        cs = acc_ref[0:1, :]
        ct = acc_ref[1:2, :]
        total_s = jnp.sum(cs)
        total_c = jnp.sum(ct)
        aux = jnp.sum(cs * ct) * _NUM_EXPERTS / (total_s * total_c)
        aux_ref[...] = aux.reshape(1, 1)


def kernel(x, W):
    wt = W.T
    scores, idx, aux = pl.pallas_call(
        _router_kernel,
        grid=(_GRID,),
        in_specs=[
            pl.BlockSpec((_BLOCK, _N_EMBD), lambda i: (i, 0)),
            pl.BlockSpec((_N_EMBD, _NUM_EXPERTS), lambda i: (0, 0)),
        ],
        out_specs=[
            pl.BlockSpec((_BLOCK, _TOP_K), lambda i: (i, 0)),
            pl.BlockSpec((_BLOCK, _TOP_K), lambda i: (i, 0)),
            pl.BlockSpec((1, 1), lambda i: (0, 0)),
        ],
        out_shape=[
            jax.ShapeDtypeStruct((_N_TOKENS, _TOP_K), jnp.float32),
            jax.ShapeDtypeStruct((_N_TOKENS, _TOP_K), jnp.int32),
            jax.ShapeDtypeStruct((1, 1), jnp.float32),
        ],
        scratch_shapes=[pltpu.VMEM((2, _NUM_EXPERTS), jnp.float32)],
    )(x, wt)
    return scores, idx, aux[0, 0]


# speedup vs baseline: 3.7948x; 1.1691x over previous
import jax
import jax.numpy as jnp
from jax.experimental import pallas as pl
from jax.experimental.pallas import tpu as pltpu

_NUM_EXPERTS = 64
_TOP_K = 2
_N_TOKENS = 16384
_N_EMBD = 2048
_BLOCK = 1024
_GRID = _N_TOKENS // _BLOCK


def _router_kernel(x_ref, wt_ref, scores_ref, idx_ref, aux_ref, acc_ref):
    step = pl.program_id(0)

    @pl.when(step == 0)
    def _init():
        acc_ref[...] = jnp.zeros_like(acc_ref)

    logits = jnp.dot(x_ref[...], wt_ref[...], preferred_element_type=jnp.float32)
    m = jnp.max(logits, axis=1, keepdims=True)
    e = jnp.exp(logits - m)
    denom = jnp.sum(e, axis=1, keepdims=True)
    s = e / denom

    iota = jax.lax.broadcasted_iota(jnp.int32, s.shape, 1)
    m1 = jnp.max(s, axis=1, keepdims=True)
    i1 = jnp.min(jnp.where(s == m1, iota, _NUM_EXPERTS), axis=1, keepdims=True)
    masked = jnp.where(iota == i1, -jnp.inf, s)
    m2 = jnp.max(masked, axis=1, keepdims=True)
    i2 = jnp.min(jnp.where(masked == m2, iota, _NUM_EXPERTS), axis=1, keepdims=True)

    scores_ref[:, 0:1] = m1
    scores_ref[:, 1:2] = m2
    idx_ref[:, 0:1] = i1
    idx_ref[:, 1:2] = i2

    colsum = jnp.sum(s, axis=0)[None, :]
    counts = jnp.sum(
        (iota == i1).astype(jnp.float32) + (iota == i2).astype(jnp.float32),
        axis=0,
    )[None, :]
    acc_ref[0:1, :] += colsum
    acc_ref[1:2, :] += counts

    @pl.when(step == _GRID - 1)
    def _finish():
        cs = acc_ref[0:1, :]
        ct = acc_ref[1:2, :]
        total_s = jnp.sum(cs)
        total_c = jnp.sum(ct)
        aux = jnp.sum(cs * ct) * _NUM_EXPERTS / (total_s * total_c)
        aux_ref[...] = aux.reshape(1, 1)


def kernel(x, W):
    wt = W.T
    scores, idx, aux = pl.pallas_call(
        _router_kernel,
        grid=(_GRID,),
        in_specs=[
            pl.BlockSpec((_BLOCK, _N_EMBD), lambda i: (i, 0)),
            pl.BlockSpec((_N_EMBD, _NUM_EXPERTS), lambda i: (0, 0)),
        ],
        out_specs=[
            pl.BlockSpec((_BLOCK, _TOP_K), lambda i: (i, 0)),
            pl.BlockSpec((_BLOCK, _TOP_K), lambda i: (i, 0)),
            pl.BlockSpec((1, 1), lambda i: (0, 0)),
        ],
        out_shape=[
            jax.ShapeDtypeStruct((_N_TOKENS, _TOP_K), jnp.float32),
            jax.ShapeDtypeStruct((_N_TOKENS, _TOP_K), jnp.int32),
            jax.ShapeDtypeStruct((1, 1), jnp.float32),
        ],
        scratch_shapes=[pltpu.VMEM((2, _NUM_EXPERTS), jnp.float32)],
    )(x, wt)
    return scores, idx, aux[0, 0]


# speedup vs baseline: 3.9586x; 1.0432x over previous
import jax
import jax.numpy as jnp
from jax.experimental import pallas as pl
from jax.experimental.pallas import tpu as pltpu

_NUM_EXPERTS = 64
_TOP_K = 2
_N_TOKENS = 16384
_N_EMBD = 2048
_BLOCK = 2048
_GRID = _N_TOKENS // _BLOCK


def _router_kernel(x_ref, wt_ref, scores_ref, idx_ref, aux_ref, acc_ref):
    step = pl.program_id(0)

    @pl.when(step == 0)
    def _init():
        acc_ref[...] = jnp.zeros_like(acc_ref)

    logits = jnp.dot(x_ref[...], wt_ref[...], preferred_element_type=jnp.float32)
    m = jnp.max(logits, axis=1, keepdims=True)
    e = jnp.exp(logits - m)
    denom = jnp.sum(e, axis=1, keepdims=True)
    s = e / denom

    iota = jax.lax.broadcasted_iota(jnp.int32, s.shape, 1)
    m1 = jnp.max(s, axis=1, keepdims=True)
    i1 = jnp.min(jnp.where(s == m1, iota, _NUM_EXPERTS), axis=1, keepdims=True)
    masked = jnp.where(iota == i1, -jnp.inf, s)
    m2 = jnp.max(masked, axis=1, keepdims=True)
    i2 = jnp.min(jnp.where(masked == m2, iota, _NUM_EXPERTS), axis=1, keepdims=True)

    scores_ref[:, 0:1] = m1
    scores_ref[:, 1:2] = m2
    idx_ref[:, 0:1] = i1
    idx_ref[:, 1:2] = i2

    colsum = jnp.sum(s, axis=0)[None, :]
    counts = jnp.sum(
        (iota == i1).astype(jnp.float32) + (iota == i2).astype(jnp.float32),
        axis=0,
    )[None, :]
    acc_ref[0:1, :] += colsum
    acc_ref[1:2, :] += counts

    @pl.when(step == _GRID - 1)
    def _finish():
        cs = acc_ref[0:1, :]
        ct = acc_ref[1:2, :]
        total_s = jnp.sum(cs)
        total_c = jnp.sum(ct)
        aux = jnp.sum(cs * ct) * _NUM_EXPERTS / (total_s * total_c)
        aux_ref[...] = aux.reshape(1, 1)


def kernel(x, W):
    wt = W.T
    scores, idx, aux = pl.pallas_call(
        _router_kernel,
        grid=(_GRID,),
        in_specs=[
            pl.BlockSpec((_BLOCK, _N_EMBD), lambda i: (i, 0)),
            pl.BlockSpec((_N_EMBD, _NUM_EXPERTS), lambda i: (0, 0)),
        ],
        out_specs=[
            pl.BlockSpec((_BLOCK, _TOP_K), lambda i: (i, 0)),
            pl.BlockSpec((_BLOCK, _TOP_K), lambda i: (i, 0)),
            pl.BlockSpec((1, 1), lambda i: (0, 0)),
        ],
        out_shape=[
            jax.ShapeDtypeStruct((_N_TOKENS, _TOP_K), jnp.float32),
            jax.ShapeDtypeStruct((_N_TOKENS, _TOP_K), jnp.int32),
            jax.ShapeDtypeStruct((1, 1), jnp.float32),
        ],
        scratch_shapes=[pltpu.VMEM((2, _NUM_EXPERTS), jnp.float32)],
    )(x, wt)
    return scores, idx, aux[0, 0]


# speedup vs baseline: 5.4558x; 1.3782x over previous
import jax
import jax.numpy as jnp
from jax.experimental import pallas as pl
from jax.experimental.pallas import tpu as pltpu

_NUM_EXPERTS = 64
_TOP_K = 2
_N_TOKENS = 16384
_N_EMBD = 2048
_BLOCK = 2048
_GRID = _N_TOKENS // _BLOCK
_LANES = 128


def _router_kernel(x_ref, wt_ref, scores_ref, idx_ref, aux_ref, accs_ref, accc_ref):
    step = pl.program_id(0)

    @pl.when(step == 0)
    def _init():
        accs_ref[...] = jnp.zeros_like(accs_ref)
        accc_ref[...] = jnp.zeros_like(accc_ref)

    logits = jnp.dot(x_ref[...], wt_ref[...], preferred_element_type=jnp.float32)
    lt = logits.T

    m = jnp.max(lt, axis=0, keepdims=True)
    iota = jax.lax.broadcasted_iota(jnp.int32, lt.shape, 0)
    i1 = jnp.min(jnp.where(lt == m, iota, _NUM_EXPERTS), axis=0, keepdims=True)
    c1 = iota == i1
    masked = jnp.where(c1, -jnp.inf, lt)
    m2 = jnp.max(masked, axis=0, keepdims=True)
    i2 = jnp.min(jnp.where(masked == m2, iota, _NUM_EXPERTS), axis=0, keepdims=True)
    c2 = iota == i2

    e = jnp.exp(lt - m)
    denom = jnp.sum(e, axis=0, keepdims=True)
    r = 1.0 / denom
    s2 = jnp.exp(m2 - m) * r
    st = e * r
    cnt = jnp.where(c1, 1.0, 0.0) + jnp.where(c2, 1.0, 0.0)

    ssum = st[:, 0:_LANES]
    csum = cnt[:, 0:_LANES]
    for k in range(1, _BLOCK // _LANES):
        ssum = ssum + st[:, k * _LANES:(k + 1) * _LANES]
        csum = csum + cnt[:, k * _LANES:(k + 1) * _LANES]
    accs_ref[...] += ssum
    accc_ref[...] += csum

    scores_ref[0:1, :] = r
    scores_ref[1:2, :] = s2
    idx_ref[0:1, :] = i1
    idx_ref[1:2, :] = i2

    @pl.when(step == _GRID - 1)
    def _finish():
        cs = jnp.sum(accs_ref[...], axis=1)
        ct = jnp.sum(accc_ref[...], axis=1)
        aux = jnp.sum(cs * ct) * _NUM_EXPERTS / (jnp.sum(cs) * jnp.sum(ct))
        aux_ref[...] = aux.reshape(1, 1)


def kernel(x, W):
    wt = W.T
    scores_t, idx_t, aux = pl.pallas_call(
        _router_kernel,
        grid=(_GRID,),
        in_specs=[
            pl.BlockSpec((_BLOCK, _N_EMBD), lambda i: (i, 0)),
            pl.BlockSpec((_N_EMBD, _NUM_EXPERTS), lambda i: (0, 0)),
        ],
        out_specs=[
            pl.BlockSpec((_TOP_K, _BLOCK), lambda i: (0, i)),
            pl.BlockSpec((_TOP_K, _BLOCK), lambda i: (0, i)),
            pl.BlockSpec((1, 1), lambda i: (0, 0)),
        ],
        out_shape=[
            jax.ShapeDtypeStruct((_TOP_K, _N_TOKENS), jnp.float32),
            jax.ShapeDtypeStruct((_TOP_K, _N_TOKENS), jnp.int32),
            jax.ShapeDtypeStruct((1, 1), jnp.float32),
        ],
        scratch_shapes=[
            pltpu.VMEM((_NUM_EXPERTS, _LANES), jnp.float32),
            pltpu.VMEM((_NUM_EXPERTS, _LANES), jnp.float32),
        ],
    )(x, wt)
    return scores_t.T, idx_t.T, aux[0, 0]


# speedup vs baseline: 5.5809x; 1.0229x over previous
import jax
import jax.numpy as jnp
from jax.experimental import pallas as pl
from jax.experimental.pallas import tpu as pltpu

_NUM_EXPERTS = 64
_TOP_K = 2
_N_TOKENS = 16384
_N_EMBD = 2048
_BLOCK = 1024
_GRID = _N_TOKENS // _BLOCK
_LANES = 128


def _router_kernel(x_ref, wt_ref, scores_ref, idx_ref, aux_ref, accs_ref, accc_ref):
    step = pl.program_id(0)

    @pl.when(step == 0)
    def _init():
        accs_ref[...] = jnp.zeros_like(accs_ref)
        accc_ref[...] = jnp.zeros_like(accc_ref)

    logits = jnp.dot(x_ref[...], wt_ref[...], preferred_element_type=jnp.float32)
    lt = logits.T

    m = jnp.max(lt, axis=0, keepdims=True)
    iota = jax.lax.broadcasted_iota(jnp.int32, lt.shape, 0)
    i1 = jnp.min(jnp.where(lt == m, iota, _NUM_EXPERTS), axis=0, keepdims=True)
    c1 = iota == i1
    masked = jnp.where(c1, -jnp.inf, lt)
    m2 = jnp.max(masked, axis=0, keepdims=True)
    i2 = jnp.min(jnp.where(masked == m2, iota, _NUM_EXPERTS), axis=0, keepdims=True)
    c2 = iota == i2

    e = jnp.exp(lt - m)
    denom = jnp.sum(e, axis=0, keepdims=True)
    r = 1.0 / denom
    s2 = jnp.exp(m2 - m) * r
    st = e * r
    cnt = jnp.where(c1, 1.0, 0.0) + jnp.where(c2, 1.0, 0.0)

    ssum = st[:, 0:_LANES]
    csum = cnt[:, 0:_LANES]
    for k in range(1, _BLOCK // _LANES):
        ssum = ssum + st[:, k * _LANES:(k + 1) * _LANES]
        csum = csum + cnt[:, k * _LANES:(k + 1) * _LANES]
    accs_ref[...] += ssum
    accc_ref[...] += csum

    scores_ref[0:1, :] = r
    scores_ref[1:2, :] = s2
    idx_ref[0:1, :] = i1
    idx_ref[1:2, :] = i2

    @pl.when(step == _GRID - 1)
    def _finish():
        cs = jnp.sum(accs_ref[...], axis=1)
        ct = jnp.sum(accc_ref[...], axis=1)
        aux = jnp.sum(cs * ct) * _NUM_EXPERTS / (jnp.sum(cs) * jnp.sum(ct))
        aux_ref[...] = aux.reshape(1, 1)


def kernel(x, W):
    wt = W.T
    scores_t, idx_t, aux = pl.pallas_call(
        _router_kernel,
        grid=(_GRID,),
        in_specs=[
            pl.BlockSpec((_BLOCK, _N_EMBD), lambda i: (i, 0)),
            pl.BlockSpec((_N_EMBD, _NUM_EXPERTS), lambda i: (0, 0)),
        ],
        out_specs=[
            pl.BlockSpec((_TOP_K, _BLOCK), lambda i: (0, i)),
            pl.BlockSpec((_TOP_K, _BLOCK), lambda i: (0, i)),
            pl.BlockSpec((1, 1), lambda i: (0, 0)),
        ],
        out_shape=[
            jax.ShapeDtypeStruct((_TOP_K, _N_TOKENS), jnp.float32),
            jax.ShapeDtypeStruct((_TOP_K, _N_TOKENS), jnp.int32),
            jax.ShapeDtypeStruct((1, 1), jnp.float32),
        ],
        scratch_shapes=[
            pltpu.VMEM((_NUM_EXPERTS, _LANES), jnp.float32),
            pltpu.VMEM((_NUM_EXPERTS, _LANES), jnp.float32),
        ],
    )(x, wt)
    return scores_t.T, idx_t.T, aux[0, 0]
